# Initial kernel scaffold; baseline (speedup 1.0000x reference)
import jax
import jax.numpy as jnp
from jax.experimental import pallas as pl


def kernel(x, W_enc, b_enc, W_dec, b_dec, K):
    raise NotImplementedError("write your pallas kernel here")


# speedup vs baseline: 5.0735x; 5.0735x over previous
import functools

import jax
import jax.numpy as jnp
from jax.experimental import pallas as pl
from jax.experimental.pallas import tpu as pltpu

FEATS = 1024
HID = 16384
NTOK = 4096

BM = 128
BH = 2048
NJ = HID // BH

BM2 = 512
BK = 4096


def _sort_key(af):
    bits = jax.lax.bitcast_convert_type(af, jnp.uint32)
    s = bits >> 31
    return bits ^ (s * jnp.uint32(0x7FFFFFFF) + jnp.uint32(0x80000000))


def _enc_kernel(k_ref, x_ref, we_ref, be_ref, bd_ref, f_ref, a_scr, u_scr):
    j = pl.program_id(1)
    xbar = x_ref[...] - bd_ref[...]
    a = jax.lax.dot_general(xbar, we_ref[...], (((1,), (1,)), ((), ())),
                            preferred_element_type=jnp.float32)
    a = a + be_ref[...]
    a_scr[j] = a
    u_scr[j] = _sort_key(a)

    @pl.when(j == NJ - 1)
    def _():
        k = k_ref[0]
        ukey = u_scr[...]

        def body(t, prefix):
            bit = jax.lax.shift_right_logical(
                jnp.uint32(0x80000000), t.astype(jnp.uint32))
            trial = prefix | bit
            cnt = jnp.sum((ukey >= trial).astype(jnp.int32),
                          axis=(0, 2), keepdims=True)
            return jnp.where(cnt >= k, trial, prefix)

        prefix = jax.lax.fori_loop(
            0, 32, body, jnp.zeros((1, BM, 1), jnp.uint32))
        thr = prefix[0]
        for jj in range(NJ):
            keep = u_scr[jj] >= thr
            f_ref[:, jj * BH:(jj + 1) * BH] = jnp.where(
                keep, jnp.maximum(a_scr[jj], 0.0), 0.0)


def _dec_kernel(f_ref, wd_ref, bd_ref, xhat_ref):
    kblk = pl.program_id(1)
    part = jax.lax.dot_general(f_ref[...], wd_ref[...],
                               (((1,), (1,)), ((), ())),
                               preferred_element_type=jnp.float32)

    @pl.when(kblk == 0)
    def _():
        xhat_ref[...] = bd_ref[...] + part

    @pl.when(kblk != 0)
    def _():
        xhat_ref[...] = xhat_ref[...] + part


@functools.partial(jax.jit, static_argnums=())
def _run(x, W_enc, b_enc, W_dec, b_dec, karr):
    be2 = b_enc.reshape(1, HID)
    bd2 = b_dec.reshape(1, FEATS)

    f = pl.pallas_call(
        _enc_kernel,
        grid_spec=pltpu.PrefetchScalarGridSpec(
            num_scalar_prefetch=1,
            grid=(NTOK // BM, NJ),
            in_specs=[
                pl.BlockSpec((BM, FEATS), lambda i, j, kk: (i, 0)),
                pl.BlockSpec((BH, FEATS), lambda i, j, kk: (j, 0)),
                pl.BlockSpec((1, BH), lambda i, j, kk: (0, j)),
                pl.BlockSpec((1, FEATS), lambda i, j, kk: (0, 0)),
            ],
            out_specs=pl.BlockSpec((BM, HID), lambda i, j, kk: (i, 0)),
            scratch_shapes=[
                pltpu.VMEM((NJ, BM, BH), jnp.float32),
                pltpu.VMEM((NJ, BM, BH), jnp.uint32),
            ],
        ),
        out_shape=jax.ShapeDtypeStruct((NTOK, HID), jnp.float32),
    )(karr, x, W_enc, be2, bd2)

    xhat = pl.pallas_call(
        _dec_kernel,
        grid=(NTOK // BM2, HID // BK),
        in_specs=[
            pl.BlockSpec((BM2, BK), lambda i, k: (i, k)),
            pl.BlockSpec((FEATS, BK), lambda i, k: (0, k)),
            pl.BlockSpec((1, FEATS), lambda i, k: (0, 0)),
        ],
        out_specs=pl.BlockSpec((BM2, FEATS), lambda i, k: (i, 0)),
        out_shape=jax.ShapeDtypeStruct((NTOK, FEATS), jnp.float32),
    )(f, W_dec, bd2)

    return xhat, f


def kernel(x, W_enc, b_enc, W_dec, b_dec, K):
    karr = jnp.full((1,), K, jnp.int32)
    return _run(x, W_enc, b_enc, W_dec, b_dec, karr)
